# Initial kernel scaffold; baseline (speedup 1.0000x reference)
import jax
import jax.numpy as jnp
from jax.experimental import pallas as pl


def kernel(x, edge_index, edge_label_index, W1, b1, W2, b2):
    raise NotImplementedError("write your pallas kernel here")


# speedup vs baseline: 22.9162x; 22.9162x over previous
import functools

import jax
import jax.numpy as jnp
from jax import lax
from jax.experimental import pallas as pl
from jax.experimental.pallas import tpu as pltpu
from jax.experimental.pallas import tpu_sc as plsc

N = 10000
E = 320000
EL = 320000
D_IN = 128
H1 = 64
H2 = 32

NC = 2
NS = 16
NW = NC * NS

NP = 10240
ROWS_W = NP // NS

EW = E // NW
CH = 80
NCH = EW // CH

NG = 4
QW = NW // NG
DG = H2 // NG
ELW = EL // QW
SCH = 2000
NSCH = ELW // SCH

_mesh = plsc.VectorSubcoreMesh(core_axis_name="c", subcore_axis_name="s")
_sc_params = pltpu.CompilerParams(use_tc_tiling_on_sc=False)


def _deg_body(dst_hbm, degp_hbm, dst_v, ones_v, bounce_v, acc):
    c = lax.axis_index("c")
    s = lax.axis_index("s")
    pltpu.sync_copy(dst_hbm.at[c, s], dst_v)
    for i in range(CH // 16):
        ones_v[pl.ds(i * 16, 16)] = jnp.ones((16,), jnp.float32)
    for i in range(ROWS_W // 16):
        bounce_v[pl.ds(i * 16, 16)] = jnp.zeros((16,), jnp.float32)
    pltpu.sync_copy(bounce_v, acc.at[pl.ds(s * ROWS_W, ROWS_W)])
    plsc.subcore_barrier()

    def body(j, carry):
        pltpu.sync_copy(ones_v, acc.at[dst_v.at[j]], add=True)
        return carry

    lax.fori_loop(0, NCH, body, 0)
    plsc.subcore_barrier()
    pltpu.sync_copy(acc.at[pl.ds(s * ROWS_W, ROWS_W)], bounce_v)
    pltpu.sync_copy(bounce_v, degp_hbm.at[c, pl.ds(s * ROWS_W, ROWS_W)])


_deg_call = pl.kernel(
    _deg_body,
    out_type=jax.ShapeDtypeStruct((NC, NP), jnp.float32),
    mesh=_mesh,
    compiler_params=_sc_params,
    scratch_types=[
        pltpu.VMEM((NCH, CH), jnp.int32),
        pltpu.VMEM((CH,), jnp.float32),
        pltpu.VMEM((ROWS_W,), jnp.float32),
        pltpu.VMEM_SHARED((NP,), jnp.float32),
    ],
)


def _scat_body(u_hbm, src_hbm, dst_hbm, zeros_hbm, accp_hbm,
               src_v, dst_v, rows_v, bounce_v, acc):
    c = lax.axis_index("c")
    s = lax.axis_index("s")
    pltpu.sync_copy(src_hbm.at[c, s], src_v)
    pltpu.sync_copy(dst_hbm.at[c, s], dst_v)
    pltpu.sync_copy(zeros_hbm.at[pl.ds(s * ROWS_W, ROWS_W)], bounce_v)
    pltpu.sync_copy(bounce_v, acc.at[pl.ds(s * ROWS_W, ROWS_W)])
    plsc.subcore_barrier()

    def body(j, carry):
        pltpu.sync_copy(u_hbm.at[src_v.at[j]], rows_v)
        pltpu.sync_copy(rows_v, acc.at[dst_v.at[j]], add=True)
        return carry

    lax.fori_loop(0, NCH, body, 0)
    plsc.subcore_barrier()
    pltpu.sync_copy(acc.at[pl.ds(s * ROWS_W, ROWS_W)], bounce_v)
    pltpu.sync_copy(bounce_v, accp_hbm.at[c, pl.ds(s * ROWS_W, ROWS_W)])


def _make_scatter(h):
    return pl.kernel(
        _scat_body,
        out_type=jax.ShapeDtypeStruct((NC, NP, h), jnp.float32),
        mesh=_mesh,
        compiler_params=_sc_params,
        scratch_types=[
            pltpu.VMEM((NCH, CH), jnp.int32),
            pltpu.VMEM((NCH, CH), jnp.int32),
            pltpu.VMEM((CH, h), jnp.float32),
            pltpu.VMEM((ROWS_W, h), jnp.float32),
            pltpu.VMEM_SHARED((NP, h), jnp.float32),
        ],
    )


_scat_call_1 = _make_scatter(H1)
_scat_call_2 = _make_scatter(H2)


def _dec_body(z2t_hbm, ea_hbm, eb_hbm, outp_hbm, cols_v, a_v, b_v, o_v):
    c = lax.axis_index("c")
    s = lax.axis_index("s")
    wid = s * NC + c
    g = wid % NG
    q = wid // NG
    pltpu.sync_copy(z2t_hbm.at[pl.ds(g * DG, DG)], cols_v)

    def outer(t, carry):
        pltpu.sync_copy(ea_hbm.at[q, t], a_v)
        pltpu.sync_copy(eb_hbm.at[q, t], b_v)

        def inner(i, icarry):
            va = a_v[pl.ds(i * 16, 16)]
            vb = b_v[pl.ds(i * 16, 16)]
            acc = jnp.zeros((16,), jnp.float32)
            for d in range(DG):
                dv = jnp.full((16,), d, jnp.int32)
                ga = plsc.load_gather(cols_v, [dv, va])
                gb = plsc.load_gather(cols_v, [dv, vb])
                acc = acc + ga * gb
            o_v[pl.ds(i * 16, 16)] = acc
            return icarry

        lax.fori_loop(0, SCH // 16, inner, 0)
        pltpu.sync_copy(o_v, outp_hbm.at[g, pl.ds(q * ELW + t * SCH, SCH)])
        return carry

    lax.fori_loop(0, NSCH, outer, 0)


_dec_call = pl.kernel(
    _dec_body,
    out_type=jax.ShapeDtypeStruct((NG, EL), jnp.float32),
    mesh=_mesh,
    compiler_params=pltpu.CompilerParams(
        use_tc_tiling_on_sc=False, needs_layout_passes=False),
    scratch_types=[
        pltpu.VMEM((DG, N), jnp.float32),
        pltpu.VMEM((SCH,), jnp.int32),
        pltpu.VMEM((SCH,), jnp.int32),
        pltpu.VMEM((SCH,), jnp.float32),
    ],
)


def _tc1_body(x_ref, w1_ref, degp_ref, u1_ref, dis_ref):
    deg = degp_ref[0] + degp_ref[1] + 1.0
    dis = lax.rsqrt(deg)
    h = jnp.dot(x_ref[...], w1_ref[...], preferred_element_type=jnp.float32)
    u1_ref[...] = h * dis
    dis_ref[...] = dis


_tc1_call = pl.pallas_call(
    _tc1_body,
    out_shape=[
        jax.ShapeDtypeStruct((N, H1), jnp.float32),
        jax.ShapeDtypeStruct((N, 1), jnp.float32),
    ],
)


def _tc2_body(accp_ref, u1_ref, dis_ref, b1_ref, w2_ref, u2_ref):
    dis = dis_ref[...]
    z1 = dis * (accp_ref[0] + accp_ref[1] + u1_ref[...]) + b1_ref[...]
    z1 = jnp.maximum(z1, 0.0)
    h2 = jnp.dot(z1, w2_ref[...], preferred_element_type=jnp.float32)
    u2_ref[...] = h2 * dis


_tc2_call = pl.pallas_call(
    _tc2_body,
    out_shape=jax.ShapeDtypeStruct((N, H2), jnp.float32),
)


def _tc3_body(accp_ref, u2_ref, dis_ref, b2_ref, z2_ref):
    z2_ref[...] = (dis_ref[...] * (accp_ref[0] + accp_ref[1] + u2_ref[...])
                   + b2_ref[...])


_tc3_call = pl.pallas_call(
    _tc3_body,
    out_shape=jax.ShapeDtypeStruct((N, H2), jnp.float32),
)


def _tc4_body(decp_ref, out_ref):
    d = decp_ref[...]
    out_ref[...] = d[0] + d[1] + d[2] + d[3]


_tc4_call = pl.pallas_call(
    _tc4_body,
    out_shape=jax.ShapeDtypeStruct((EL,), jnp.float32),
)


def kernel(x, edge_index, edge_label_index, W1, b1, W2, b2):
    src4 = edge_index[0].reshape(NC, NS, NCH, CH)
    dst4 = edge_index[1].reshape(NC, NS, NCH, CH)
    ea = edge_label_index[0].reshape(QW, NSCH, SCH)
    eb = edge_label_index[1].reshape(QW, NSCH, SCH)

    degp = _deg_call(dst4)
    degp2 = degp[:, :N, None]
    u1, dis = _tc1_call(x, W1, degp2)

    accp1 = _scat_call_1(u1, src4, dst4, jnp.zeros((NP, H1), jnp.float32))
    u2 = _tc2_call(accp1[:, :N, :], u1, dis, b1, W2)

    accp2 = _scat_call_2(u2, src4, dst4, jnp.zeros((NP, H2), jnp.float32))
    z2 = _tc3_call(accp2[:, :N, :], u2, dis, b2)

    z2t = jnp.transpose(z2)
    decp = _dec_call(z2t, ea, eb)
    return _tc4_call(decp)


# speedup vs baseline: 31.2679x; 1.3644x over previous
import functools

import jax
import jax.numpy as jnp
from jax import lax
from jax.experimental import pallas as pl
from jax.experimental.pallas import tpu as pltpu
from jax.experimental.pallas import tpu_sc as plsc

N = 10000
E = 320000
EL = 320000
D_IN = 128
H1 = 64
H2 = 32

NC = 2
NS = 16
NW = NC * NS

NP = 10240
ROWS_W = NP // NS

EW = E // NW
CH = 80
NCH = EW // CH
PAIRS = NCH // 2

NG = 4
QW = NW // NG
DG = H2 // NG
ELW = EL // QW
SCH = 2000
NSCH = ELW // SCH

_mesh = plsc.VectorSubcoreMesh(core_axis_name="c", subcore_axis_name="s")
_sc_params = pltpu.CompilerParams(use_tc_tiling_on_sc=False)


def _deg_body(dst_hbm, degp_hbm, dst_v, ones_v, bounce_v, acc, s0, s1):
    c = lax.axis_index("c")
    s = lax.axis_index("s")
    pltpu.sync_copy(dst_hbm.at[c, s], dst_v)
    for i in range(CH // 16):
        ones_v[pl.ds(i * 16, 16)] = jnp.ones((16,), jnp.float32)
    for i in range(ROWS_W // 16):
        bounce_v[pl.ds(i * 16, 16)] = jnp.zeros((16,), jnp.float32)
    pltpu.sync_copy(bounce_v, acc.at[pl.ds(s * ROWS_W, ROWS_W)])
    plsc.subcore_barrier()

    d0 = pltpu.async_copy(ones_v, acc.at[dst_v.at[0]], s0, add=True)
    d1 = pltpu.async_copy(ones_v, acc.at[dst_v.at[1]], s1, add=True)

    def body(g, carry):
        pltpu.async_copy(ones_v, acc.at[dst_v.at[2 * g]], s0, add=True).wait()
        pltpu.async_copy(ones_v, acc.at[dst_v.at[2 * g + 1]], s1,
                         add=True).wait()
        return carry

    lax.fori_loop(1, PAIRS, body, 0)
    pltpu.async_copy(ones_v, acc.at[dst_v.at[NCH - 1]], s0, add=True).wait()
    d0.wait()
    d1.wait()
    plsc.subcore_barrier()
    pltpu.sync_copy(acc.at[pl.ds(s * ROWS_W, ROWS_W)], bounce_v)
    pltpu.sync_copy(bounce_v, degp_hbm.at[c, pl.ds(s * ROWS_W, ROWS_W)])


_deg_call = pl.kernel(
    _deg_body,
    out_type=jax.ShapeDtypeStruct((NC, NP), jnp.float32),
    mesh=_mesh,
    compiler_params=_sc_params,
    scratch_types=[
        pltpu.VMEM((NCH, CH), jnp.int32),
        pltpu.VMEM((CH,), jnp.float32),
        pltpu.VMEM((ROWS_W,), jnp.float32),
        pltpu.VMEM_SHARED((NP,), jnp.float32),
        pltpu.SemaphoreType.DMA,
        pltpu.SemaphoreType.DMA,
    ],
)


def _scat_body(u_hbm, src_hbm, dst_hbm, zeros_hbm, accp_hbm,
               src_v, dst_v, rows0, rows1, bounce_v, acc, g0, g1, t0, t1):
    c = lax.axis_index("c")
    s = lax.axis_index("s")
    pltpu.sync_copy(src_hbm.at[c, s], src_v)
    pltpu.sync_copy(dst_hbm.at[c, s], dst_v)
    pltpu.sync_copy(zeros_hbm.at[pl.ds(s * ROWS_W, ROWS_W)], bounce_v)
    pltpu.sync_copy(bounce_v, acc.at[pl.ds(s * ROWS_W, ROWS_W)])
    plsc.subcore_barrier()

    pltpu.async_copy(u_hbm.at[src_v.at[0]], rows0, g0)
    pltpu.async_copy(u_hbm.at[src_v.at[1]], rows1, g1)

    def pair(g, pf0, pf1):
        j0 = 2 * g
        pltpu.make_async_copy(u_hbm.at[src_v.at[j0]], rows0, g0).wait()
        d0 = pltpu.async_copy(rows0, acc.at[dst_v.at[j0]], t0, add=True)
        pltpu.make_async_copy(u_hbm.at[src_v.at[j0 + 1]], rows1, g1).wait()
        d1 = pltpu.async_copy(rows1, acc.at[dst_v.at[j0 + 1]], t1, add=True)
        d0.wait()
        if pf0:
            pltpu.async_copy(u_hbm.at[src_v.at[j0 + 2]], rows0, g0)
        d1.wait()
        if pf1:
            pltpu.async_copy(u_hbm.at[src_v.at[j0 + 3]], rows1, g1)

    def body(g, carry):
        pair(g, True, True)
        return carry

    lax.fori_loop(0, PAIRS - 1, body, 0)
    pair(PAIRS - 1, True, False)
    pltpu.make_async_copy(u_hbm.at[src_v.at[NCH - 1]], rows0, g0).wait()
    pltpu.async_copy(rows0, acc.at[dst_v.at[NCH - 1]], t0, add=True).wait()
    plsc.subcore_barrier()
    pltpu.sync_copy(acc.at[pl.ds(s * ROWS_W, ROWS_W)], bounce_v)
    pltpu.sync_copy(bounce_v, accp_hbm.at[c, pl.ds(s * ROWS_W, ROWS_W)])


def _make_scatter(h):
    return pl.kernel(
        _scat_body,
        out_type=jax.ShapeDtypeStruct((NC, NP, h), jnp.float32),
        mesh=_mesh,
        compiler_params=_sc_params,
        scratch_types=[
            pltpu.VMEM((NCH, CH), jnp.int32),
            pltpu.VMEM((NCH, CH), jnp.int32),
            pltpu.VMEM((CH, h), jnp.float32),
            pltpu.VMEM((CH, h), jnp.float32),
            pltpu.VMEM((ROWS_W, h), jnp.float32),
            pltpu.VMEM_SHARED((NP, h), jnp.float32),
            pltpu.SemaphoreType.DMA,
            pltpu.SemaphoreType.DMA,
            pltpu.SemaphoreType.DMA,
            pltpu.SemaphoreType.DMA,
        ],
    )


_scat_call_1 = _make_scatter(H1)
_scat_call_2 = _make_scatter(H2)


def _dec_body(z2t_hbm, ea_hbm, eb_hbm, outp_hbm, cols_v,
              a0, a1, b0, b1, o0, o1, sa0, sa1, sb0, sb1, so0, so1):
    c = lax.axis_index("c")
    s = lax.axis_index("s")
    wid = s * NC + c
    gd = wid % NG
    q = wid // NG
    pltpu.sync_copy(z2t_hbm.at[pl.ds(gd * DG, DG)], cols_v)

    def inner_loop(a_v, b_v, o_v):
        def inner(i, icarry):
            va = a_v[pl.ds(i * 16, 16)]
            vb = b_v[pl.ds(i * 16, 16)]
            acc = jnp.zeros((16,), jnp.float32)
            for d in range(DG):
                dv = jnp.full((16,), d, jnp.int32)
                ga = plsc.load_gather(cols_v, [dv, va])
                gb = plsc.load_gather(cols_v, [dv, vb])
                acc = acc + ga * gb
            o_v[pl.ds(i * 16, 16)] = acc
            return icarry

        lax.fori_loop(0, SCH // 16, inner, 0)

    def out_slice(t):
        return outp_hbm.at[gd, pl.ds(q * ELW + t * SCH, SCH)]

    def half(t, a_v, b_v, o_v, sa, sb, so, wait_wb, prefetch):
        pltpu.make_async_copy(ea_hbm.at[q, t], a_v, sa).wait()
        pltpu.make_async_copy(eb_hbm.at[q, t], b_v, sb).wait()
        if wait_wb:
            pltpu.make_async_copy(o_v, out_slice(t), so).wait()
        inner_loop(a_v, b_v, o_v)
        pltpu.async_copy(o_v, out_slice(t), so)
        if prefetch:
            pltpu.async_copy(ea_hbm.at[q, t + 2], a_v, sa)
            pltpu.async_copy(eb_hbm.at[q, t + 2], b_v, sb)

    def pair(g, wait_wb, prefetch):
        half(2 * g, a0, b0, o0, sa0, sb0, so0, wait_wb, prefetch)
        half(2 * g + 1, a1, b1, o1, sa1, sb1, so1, wait_wb, prefetch)

    pltpu.async_copy(ea_hbm.at[q, 0], a0, sa0)
    pltpu.async_copy(eb_hbm.at[q, 0], b0, sb0)
    pltpu.async_copy(ea_hbm.at[q, 1], a1, sa1)
    pltpu.async_copy(eb_hbm.at[q, 1], b1, sb1)

    pair(0, False, True)

    def body(g, carry):
        pair(g, True, True)
        return carry

    lax.fori_loop(1, NSCH // 2 - 1, body, 0)
    pair(NSCH // 2 - 1, True, False)
    pltpu.make_async_copy(o0, out_slice(NSCH - 2), so0).wait()
    pltpu.make_async_copy(o1, out_slice(NSCH - 1), so1).wait()


_dec_call = pl.kernel(
    _dec_body,
    out_type=jax.ShapeDtypeStruct((NG, EL), jnp.float32),
    mesh=_mesh,
    compiler_params=pltpu.CompilerParams(
        use_tc_tiling_on_sc=False, needs_layout_passes=False),
    scratch_types=[
        pltpu.VMEM((DG, N), jnp.float32),
        pltpu.VMEM((SCH,), jnp.int32),
        pltpu.VMEM((SCH,), jnp.int32),
        pltpu.VMEM((SCH,), jnp.int32),
        pltpu.VMEM((SCH,), jnp.int32),
        pltpu.VMEM((SCH,), jnp.float32),
        pltpu.VMEM((SCH,), jnp.float32),
        pltpu.SemaphoreType.DMA,
        pltpu.SemaphoreType.DMA,
        pltpu.SemaphoreType.DMA,
        pltpu.SemaphoreType.DMA,
        pltpu.SemaphoreType.DMA,
        pltpu.SemaphoreType.DMA,
    ],
)


def _tc1_body(x_ref, w1_ref, degp_ref, u1_ref, dis_ref):
    deg = degp_ref[0] + degp_ref[1] + 1.0
    dis = lax.rsqrt(deg)
    h = jnp.dot(x_ref[...], w1_ref[...], preferred_element_type=jnp.float32)
    u1_ref[...] = h * dis
    dis_ref[...] = dis


_tc1_call = pl.pallas_call(
    _tc1_body,
    out_shape=[
        jax.ShapeDtypeStruct((N, H1), jnp.float32),
        jax.ShapeDtypeStruct((N, 1), jnp.float32),
    ],
)


def _tc2_body(accp_ref, u1_ref, dis_ref, b1_ref, w2_ref, u2_ref):
    dis = dis_ref[...]
    z1 = dis * (accp_ref[0] + accp_ref[1] + u1_ref[...]) + b1_ref[...]
    z1 = jnp.maximum(z1, 0.0)
    h2 = jnp.dot(z1, w2_ref[...], preferred_element_type=jnp.float32)
    u2_ref[...] = h2 * dis


_tc2_call = pl.pallas_call(
    _tc2_body,
    out_shape=jax.ShapeDtypeStruct((N, H2), jnp.float32),
)


def _tc3_body(accp_ref, u2_ref, dis_ref, b2_ref, z2_ref):
    z2_ref[...] = (dis_ref[...] * (accp_ref[0] + accp_ref[1] + u2_ref[...])
                   + b2_ref[...])


_tc3_call = pl.pallas_call(
    _tc3_body,
    out_shape=jax.ShapeDtypeStruct((N, H2), jnp.float32),
)


def _tc4_body(decp_ref, out_ref):
    d = decp_ref[...]
    out_ref[...] = d[0] + d[1] + d[2] + d[3]


_tc4_call = pl.pallas_call(
    _tc4_body,
    out_shape=jax.ShapeDtypeStruct((EL,), jnp.float32),
)


def kernel(x, edge_index, edge_label_index, W1, b1, W2, b2):
    src4 = edge_index[0].reshape(NC, NS, NCH, CH)
    dst4 = edge_index[1].reshape(NC, NS, NCH, CH)
    ea = edge_label_index[0].reshape(QW, NSCH, SCH)
    eb = edge_label_index[1].reshape(QW, NSCH, SCH)

    degp = _deg_call(dst4)
    degp2 = degp[:, :N, None]
    u1, dis = _tc1_call(x, W1, degp2)

    accp1 = _scat_call_1(u1, src4, dst4, jnp.zeros((NP, H1), jnp.float32))
    u2 = _tc2_call(accp1[:, :N, :], u1, dis, b1, W2)

    accp2 = _scat_call_2(u2, src4, dst4, jnp.zeros((NP, H2), jnp.float32))
    z2 = _tc3_call(accp2[:, :N, :], u2, dis, b2)

    z2t = jnp.transpose(z2)
    decp = _dec_call(z2t, ea, eb)
    return _tc4_call(decp)


# speedup vs baseline: 37.0061x; 1.1835x over previous
import functools

import jax
import jax.numpy as jnp
from jax import lax
from jax.experimental import pallas as pl
from jax.experimental.pallas import tpu as pltpu
from jax.experimental.pallas import tpu_sc as plsc

N = 10000
E = 320000
EL = 320000
D_IN = 128
H1 = 64
H2 = 32

NC = 2
NS = 16
NW = NC * NS

NP = 10240
ROWS_W = NP // NS

EW = E // NW
CH = 80
NCH = EW // CH
PAIRS = NCH // 2

NG = 4
QW = NW // NG
DG = H2 // NG
ELW = EL // QW
SCH = 2000
NSCH = ELW // SCH

_mesh = plsc.VectorSubcoreMesh(core_axis_name="c", subcore_axis_name="s")
_sc_params = pltpu.CompilerParams(use_tc_tiling_on_sc=False)


def _deg_body(dst_hbm, degp_hbm, dst_v, ones_v, bounce_v, acc, s0, s1):
    c = lax.axis_index("c")
    s = lax.axis_index("s")
    pltpu.sync_copy(dst_hbm.at[c, s], dst_v)
    for i in range(CH // 16):
        ones_v[pl.ds(i * 16, 16)] = jnp.ones((16,), jnp.float32)
    for i in range(ROWS_W // 16):
        bounce_v[pl.ds(i * 16, 16)] = jnp.zeros((16,), jnp.float32)
    pltpu.sync_copy(bounce_v, acc.at[pl.ds(s * ROWS_W, ROWS_W)])
    plsc.subcore_barrier()

    d0 = pltpu.async_copy(ones_v, acc.at[dst_v.at[0]], s0, add=True)
    d1 = pltpu.async_copy(ones_v, acc.at[dst_v.at[1]], s1, add=True)

    def body(g, carry):
        pltpu.async_copy(ones_v, acc.at[dst_v.at[2 * g]], s0, add=True).wait()
        pltpu.async_copy(ones_v, acc.at[dst_v.at[2 * g + 1]], s1,
                         add=True).wait()
        return carry

    lax.fori_loop(1, PAIRS, body, 0)
    pltpu.async_copy(ones_v, acc.at[dst_v.at[NCH - 1]], s0, add=True).wait()
    d0.wait()
    d1.wait()
    plsc.subcore_barrier()
    pltpu.sync_copy(acc.at[pl.ds(s * ROWS_W, ROWS_W)], bounce_v)
    pltpu.sync_copy(bounce_v, degp_hbm.at[c, pl.ds(s * ROWS_W, ROWS_W)])


_deg_call = pl.kernel(
    _deg_body,
    out_type=jax.ShapeDtypeStruct((NC, NP), jnp.float32),
    mesh=_mesh,
    compiler_params=_sc_params,
    scratch_types=[
        pltpu.VMEM((NCH, CH), jnp.int32),
        pltpu.VMEM((CH,), jnp.float32),
        pltpu.VMEM((ROWS_W,), jnp.float32),
        pltpu.VMEM_SHARED((NP,), jnp.float32),
        pltpu.SemaphoreType.DMA,
        pltpu.SemaphoreType.DMA,
    ],
)


NBUF = 4
QUADS = NCH // NBUF


def _scat_body(u_hbm, src_hbm, dst_hbm, zeros_hbm, accp_hbm,
               src_v, dst_v, rows0, rows1, rows2, rows3, bounce_v, acc,
               g0, g1, g2, g3, t0, t1, t2, t3):
    c = lax.axis_index("c")
    s = lax.axis_index("s")
    rows = (rows0, rows1, rows2, rows3)
    gsem = (g0, g1, g2, g3)
    tsem = (t0, t1, t2, t3)
    pltpu.sync_copy(src_hbm.at[c, s], src_v)
    pltpu.sync_copy(dst_hbm.at[c, s], dst_v)
    pltpu.sync_copy(zeros_hbm.at[pl.ds(s * ROWS_W, ROWS_W)], bounce_v)
    pltpu.sync_copy(bounce_v, acc.at[pl.ds(s * ROWS_W, ROWS_W)])
    plsc.subcore_barrier()

    for b in range(NBUF):
        pltpu.async_copy(u_hbm.at[src_v.at[b]], rows[b], gsem[b])

    def quad(g, prefetch):
        j0 = NBUF * g
        ds = []
        for b in range(NBUF):
            pltpu.make_async_copy(
                u_hbm.at[src_v.at[j0 + b]], rows[b], gsem[b]).wait()
            ds.append(pltpu.async_copy(
                rows[b], acc.at[dst_v.at[j0 + b]], tsem[b], add=True))
        for b in range(NBUF):
            ds[b].wait()
            if prefetch:
                pltpu.async_copy(
                    u_hbm.at[src_v.at[j0 + NBUF + b]], rows[b], gsem[b])

    def body(g, carry):
        quad(g, True)
        return carry

    lax.fori_loop(0, QUADS - 1, body, 0)
    quad(QUADS - 1, False)
    pltpu.async_copy(u_hbm.at[src_v.at[NCH - 1]], rows0, g0).wait()
    pltpu.async_copy(rows0, acc.at[dst_v.at[NCH - 1]], t0, add=True).wait()
    plsc.subcore_barrier()
    pltpu.sync_copy(acc.at[pl.ds(s * ROWS_W, ROWS_W)], bounce_v)
    pltpu.sync_copy(bounce_v, accp_hbm.at[c, pl.ds(s * ROWS_W, ROWS_W)])


def _make_scatter(h):
    return pl.kernel(
        _scat_body,
        out_type=jax.ShapeDtypeStruct((NC, NP, h), jnp.float32),
        mesh=_mesh,
        compiler_params=_sc_params,
        scratch_types=(
            [pltpu.VMEM((NCH, CH), jnp.int32),
             pltpu.VMEM((NCH, CH), jnp.int32)]
            + [pltpu.VMEM((CH, h), jnp.float32)] * NBUF
            + [pltpu.VMEM((ROWS_W, h), jnp.float32),
               pltpu.VMEM_SHARED((NP, h), jnp.float32)]
            + [pltpu.SemaphoreType.DMA] * (2 * NBUF)
        ),
    )


_scat_call_1 = _make_scatter(H1)
_scat_call_2 = _make_scatter(H2)


def _dec_body(z2t_hbm, ea_hbm, eb_hbm, outp_hbm, cols_v,
              a0, a1, b0, b1, o0, o1, sa0, sa1, sb0, sb1, so0, so1):
    c = lax.axis_index("c")
    s = lax.axis_index("s")
    wid = s * NC + c
    gd = wid % NG
    q = wid // NG
    pltpu.sync_copy(z2t_hbm.at[pl.ds(gd * DG, DG)], cols_v)

    def inner_loop(a_v, b_v, o_v):
        def inner(i, icarry):
            va = a_v[pl.ds(i * 16, 16)]
            vb = b_v[pl.ds(i * 16, 16)]
            acc = jnp.zeros((16,), jnp.float32)
            for d in range(DG):
                dv = jnp.full((16,), d, jnp.int32)
                ga = plsc.load_gather(cols_v, [dv, va])
                gb = plsc.load_gather(cols_v, [dv, vb])
                acc = acc + ga * gb
            o_v[pl.ds(i * 16, 16)] = acc
            return icarry

        lax.fori_loop(0, SCH // 16, inner, 0)

    def out_slice(t):
        return outp_hbm.at[gd, pl.ds(q * ELW + t * SCH, SCH)]

    def half(t, a_v, b_v, o_v, sa, sb, so, wait_wb, prefetch):
        pltpu.make_async_copy(ea_hbm.at[q, t], a_v, sa).wait()
        pltpu.make_async_copy(eb_hbm.at[q, t], b_v, sb).wait()
        if wait_wb:
            pltpu.make_async_copy(o_v, out_slice(t), so).wait()
        inner_loop(a_v, b_v, o_v)
        pltpu.async_copy(o_v, out_slice(t), so)
        if prefetch:
            pltpu.async_copy(ea_hbm.at[q, t + 2], a_v, sa)
            pltpu.async_copy(eb_hbm.at[q, t + 2], b_v, sb)

    def pair(g, wait_wb, prefetch):
        half(2 * g, a0, b0, o0, sa0, sb0, so0, wait_wb, prefetch)
        half(2 * g + 1, a1, b1, o1, sa1, sb1, so1, wait_wb, prefetch)

    pltpu.async_copy(ea_hbm.at[q, 0], a0, sa0)
    pltpu.async_copy(eb_hbm.at[q, 0], b0, sb0)
    pltpu.async_copy(ea_hbm.at[q, 1], a1, sa1)
    pltpu.async_copy(eb_hbm.at[q, 1], b1, sb1)

    pair(0, False, True)

    def body(g, carry):
        pair(g, True, True)
        return carry

    lax.fori_loop(1, NSCH // 2 - 1, body, 0)
    pair(NSCH // 2 - 1, True, False)
    pltpu.make_async_copy(o0, out_slice(NSCH - 2), so0).wait()
    pltpu.make_async_copy(o1, out_slice(NSCH - 1), so1).wait()


_dec_call = pl.kernel(
    _dec_body,
    out_type=jax.ShapeDtypeStruct((NG, EL), jnp.float32),
    mesh=_mesh,
    compiler_params=pltpu.CompilerParams(
        use_tc_tiling_on_sc=False, needs_layout_passes=False),
    scratch_types=[
        pltpu.VMEM((DG, N), jnp.float32),
        pltpu.VMEM((SCH,), jnp.int32),
        pltpu.VMEM((SCH,), jnp.int32),
        pltpu.VMEM((SCH,), jnp.int32),
        pltpu.VMEM((SCH,), jnp.int32),
        pltpu.VMEM((SCH,), jnp.float32),
        pltpu.VMEM((SCH,), jnp.float32),
        pltpu.SemaphoreType.DMA,
        pltpu.SemaphoreType.DMA,
        pltpu.SemaphoreType.DMA,
        pltpu.SemaphoreType.DMA,
        pltpu.SemaphoreType.DMA,
        pltpu.SemaphoreType.DMA,
    ],
)


def _tc1_body(x_ref, w1_ref, degp_ref, u1_ref, dis_ref):
    deg = degp_ref[0] + degp_ref[1] + 1.0
    dis = lax.rsqrt(deg)
    h = jnp.dot(x_ref[...], w1_ref[...], preferred_element_type=jnp.float32)
    u1_ref[...] = h * dis
    dis_ref[...] = dis


_tc1_call = pl.pallas_call(
    _tc1_body,
    out_shape=[
        jax.ShapeDtypeStruct((N, H1), jnp.float32),
        jax.ShapeDtypeStruct((N, 1), jnp.float32),
    ],
)


def _tc2_body(accp_ref, u1_ref, dis_ref, b1_ref, w2_ref, u2_ref):
    dis = dis_ref[...]
    z1 = dis * (accp_ref[0] + accp_ref[1] + u1_ref[...]) + b1_ref[...]
    z1 = jnp.maximum(z1, 0.0)
    h2 = jnp.dot(z1, w2_ref[...], preferred_element_type=jnp.float32)
    u2_ref[...] = h2 * dis


_tc2_call = pl.pallas_call(
    _tc2_body,
    out_shape=jax.ShapeDtypeStruct((N, H2), jnp.float32),
)


def _tc3_body(accp_ref, u2_ref, dis_ref, b2_ref, z2_ref):
    z2_ref[...] = (dis_ref[...] * (accp_ref[0] + accp_ref[1] + u2_ref[...])
                   + b2_ref[...])


_tc3_call = pl.pallas_call(
    _tc3_body,
    out_shape=jax.ShapeDtypeStruct((N, H2), jnp.float32),
)


def _tc4_body(decp_ref, out_ref):
    d = decp_ref[...]
    out_ref[...] = d[0] + d[1] + d[2] + d[3]


_tc4_call = pl.pallas_call(
    _tc4_body,
    out_shape=jax.ShapeDtypeStruct((EL,), jnp.float32),
)


def kernel(x, edge_index, edge_label_index, W1, b1, W2, b2):
    src4 = edge_index[0].reshape(NC, NS, NCH, CH)
    dst4 = edge_index[1].reshape(NC, NS, NCH, CH)
    ea = edge_label_index[0].reshape(QW, NSCH, SCH)
    eb = edge_label_index[1].reshape(QW, NSCH, SCH)

    degp = _deg_call(dst4)
    degp2 = degp[:, :N, None]
    u1, dis = _tc1_call(x, W1, degp2)

    accp1 = _scat_call_1(u1, src4, dst4, jnp.zeros((NP, H1), jnp.float32))
    u2 = _tc2_call(accp1[:, :N, :], u1, dis, b1, W2)

    accp2 = _scat_call_2(u2, src4, dst4, jnp.zeros((NP, H2), jnp.float32))
    z2 = _tc3_call(accp2[:, :N, :], u2, dis, b2)

    z2t = jnp.transpose(z2)
    decp = _dec_call(z2t, ea, eb)
    return _tc4_call(decp)
